# baseline (speedup 1.0000x reference)
import functools

import jax
import jax.numpy as jnp
from jax import lax
from jax.experimental import pallas as pl
from jax.experimental.pallas import tpu as pltpu
from jax.experimental.pallas import tpu_sc as plsc

_SC_CORES = 2
_SC_SUBCORES = 16


def _routing_metadata(selected_experts, routing_weights, E, B, NB):
    T, K = selected_experts.shape
    S = T * K
    flat_e = selected_experts.reshape(-1)
    gates = routing_weights.reshape(-1).astype(jnp.float32)
    onehot = (flat_e[:, None] == jnp.arange(E, dtype=jnp.int32)[None, :])
    counts = jnp.sum(onehot.astype(jnp.int32), axis=0)
    pc = ((counts + B - 1) // B) * B
    ends = jnp.cumsum(pc)
    starts = ends - pc
    rank_all = jnp.cumsum(onehot.astype(jnp.int32), axis=0) - 1
    rank = jnp.take_along_axis(rank_all, flat_e[:, None], axis=1)[:, 0]
    pos = starts[flat_e] + rank
    token_ids = (jnp.arange(S, dtype=jnp.int32) // K)
    NP = NB * B
    tok_padded = jnp.zeros((NP,), jnp.int32).at[pos].set(token_ids)
    gate_padded = jnp.zeros((NP,), jnp.float32).at[pos].set(gates)
    brow = jnp.arange(NB, dtype=jnp.int32) * B
    block_expert = jnp.sum((brow[:, None] >= ends[None, :]).astype(jnp.int32),
                           axis=1)
    block_expert = jnp.minimum(block_expert, E - 1)
    nact = ends[E - 1] // B
    block_valid = (jnp.arange(NB, dtype=jnp.int32) < nact).astype(jnp.int32)
    block_clamped = jnp.minimum(jnp.arange(NB, dtype=jnp.int32), nact - 1)
    pos_tk = pos.reshape(T, K)
    return tok_padded, gate_padded, block_expert, block_valid, block_clamped, pos_tk


def _sc_scatter(xf, pint3, NP):
    T, H = xf.shape
    NW = _SC_CORES * _SC_SUBCORES
    TW = T // NW
    K = pint3.shape[1]
    mesh = plsc.VectorSubcoreMesh(core_axis_name="c", subcore_axis_name="s")

    @functools.partial(
        pl.kernel, mesh=mesh,
        out_type=jax.ShapeDtypeStruct((NP, H), jnp.float32),
        scratch_types=[
            pltpu.VMEM((K, TW), jnp.int32),
            pltpu.VMEM((TW, H), jnp.float32),
        ] + [pltpu.SemaphoreType.DMA] * K,
    )
    def k(x_hbm, pint_hbm, out_hbm, idx_v, src_v, *sems):
        wid = lax.axis_index("s") * _SC_CORES + lax.axis_index("c")
        base = wid * TW
        pltpu.sync_copy(pint_hbm.at[wid], idx_v)
        pltpu.sync_copy(x_hbm.at[pl.ds(base, TW)], src_v)
        descs = [
            pltpu.async_copy(src_v, out_hbm.at[idx_v.at[kk]], sems[kk])
            for kk in range(K)
        ]
        for d in descs:
            d.wait()

    return k(xf, pint3)


def _sc_combine(ys, pint, T):
    H = ys.shape[1]
    NW = _SC_CORES * _SC_SUBCORES
    tok_w = T // NW
    CT = 16
    n_ch = tok_w // CT
    mesh = plsc.VectorSubcoreMesh(core_axis_name="c", subcore_axis_name="s")

    @functools.partial(
        pl.kernel, mesh=mesh,
        out_type=jax.ShapeDtypeStruct((T, H), jnp.float32),
        scratch_types=[
            pltpu.VMEM((2 * tok_w,), jnp.int32),
            pltpu.VMEM((2 * CT, H), jnp.float32),
            pltpu.VMEM((2 * CT, H), jnp.float32),
            pltpu.VMEM((CT, H), jnp.float32),
            pltpu.SemaphoreType.DMA,
            pltpu.SemaphoreType.DMA,
        ],
    )
    def k(ys_hbm, pint_hbm, y_hbm, idx_v, bufa, bufb, outb, sem0, sem1):
        wid = lax.axis_index("s") * _SC_CORES + lax.axis_index("c")
        tok_base = wid * tok_w
        pltpu.sync_copy(pint_hbm.at[pl.ds(2 * tok_base, 2 * tok_w)], idx_v)
        bufs = (bufa, bufb)
        sems = (sem0, sem1)
        descs = [None] * n_ch
        descs[0] = pltpu.async_copy(
            ys_hbm.at[idx_v.at[pl.ds(0, 2 * CT)]], bufs[0], sems[0])
        nvec = H // 16
        for c in range(n_ch):
            if c + 1 < n_ch:
                descs[c + 1] = pltpu.async_copy(
                    ys_hbm.at[idx_v.at[pl.ds((c + 1) * 2 * CT, 2 * CT)]],
                    bufs[(c + 1) % 2], sems[(c + 1) % 2])
            descs[c].wait()
            buf = bufs[c % 2]

            def body(t, _, buf=buf):
                for j in range(nvec):
                    off = j * 16
                    outb[t, pl.ds(off, 16)] = (
                        buf[2 * t, pl.ds(off, 16)]
                        + buf[2 * t + 1, pl.ds(off, 16)])
                return 0

            lax.fori_loop(0, CT, body, 0)
            pltpu.sync_copy(outb, y_hbm.at[pl.ds(tok_base + c * CT, CT)])

    return k(ys, pint)


def _gmm_body(NF, be_ref, bv_ref, bc_ref, xs_ref, w1h_ref, w1g_ref, w2_ref,
              g_ref, out_ref):
    b = pl.program_id(0)
    f = pl.program_id(1)

    @pl.when(bv_ref[b] > 0)
    def _():
        @pl.when(f == 0)
        def _():
            out_ref[...] = jnp.zeros_like(out_ref)

        x = xs_ref[...]
        h = lax.dot_general(x, w1h_ref[0], (((1,), (1,)), ((), ())),
                            preferred_element_type=jnp.float32)
        g = lax.dot_general(x, w1g_ref[0], (((1,), (1,)), ((), ())),
                            preferred_element_type=jnp.float32)
        hg = h * (g * jax.nn.sigmoid(g))
        yp = lax.dot_general(hg, w2_ref[0], (((1,), (1,)), ((), ())),
                             preferred_element_type=jnp.float32)
        out_ref[...] += yp

        @pl.when(f == NF - 1)
        def _():
            out_ref[...] *= g_ref[0, 0][:, None]


def _grouped_mlp(xs, w1, w2, gate_padded, block_expert, block_valid,
                 block_clamped, B, TF, interpret=False):
    E, F2, H = w1.shape
    F = F2 // 2
    NF = F // TF
    NP = xs.shape[0]
    NB = NP // B
    gates3 = gate_padded.reshape(NB, 1, B)

    def f_eff(f, bv, b):
        return bv[b] * f + (1 - bv[b]) * (NF - 1)

    grid_spec = pltpu.PrefetchScalarGridSpec(
        num_scalar_prefetch=3,
        grid=(NB, NF),
        in_specs=[
            pl.BlockSpec((B, H), lambda b, f, be, bv, bc: (bc[b], 0)),
            pl.BlockSpec((1, TF, H),
                         lambda b, f, be, bv, bc:
                         (be[bc[b]], f_eff(f, bv, b), 0)),
            pl.BlockSpec((1, TF, H),
                         lambda b, f, be, bv, bc:
                         (be[bc[b]], NF + f_eff(f, bv, b), 0)),
            pl.BlockSpec((1, H, TF),
                         lambda b, f, be, bv, bc:
                         (be[bc[b]], 0, f_eff(f, bv, b))),
            pl.BlockSpec((1, 1, B), lambda b, f, be, bv, bc: (bc[b], 0, 0)),
        ],
        out_specs=pl.BlockSpec((B, H), lambda b, f, be, bv, bc: (bc[b], 0)),
    )
    return pl.pallas_call(
        functools.partial(_gmm_body, NF),
        grid_spec=grid_spec,
        out_shape=jax.ShapeDtypeStruct((NP, H), jnp.float32),
        compiler_params=pltpu.CompilerParams(
            dimension_semantics=("arbitrary", "arbitrary")),
        interpret=interpret,
    )(block_expert, block_valid, block_clamped, xs, w1, w1, w2, gates3)


def _fused_experts(x, routing_weights, selected_experts, W1, W2,
                   interpret=False):
    x_shape = x.shape
    H = x_shape[-1]
    xf = x.reshape(-1, H)
    T, K = selected_experts.shape
    E = W1.shape[0]
    B = 1024
    S = T * K
    NB = (S + E * (B - 1) + B - 1) // B
    tok_padded, gate_padded, block_expert, block_valid, block_clamped, pos_tk = \
        _routing_metadata(selected_experts, routing_weights, E, B, NB)
    NW = _SC_CORES * _SC_SUBCORES
    if interpret:
        xs = jnp.take(xf, tok_padded, axis=0)
    else:
        pint3 = pos_tk.reshape(NW, T // NW, K).transpose(0, 2, 1)
        xs = _sc_scatter(xf, pint3, NB * B)
    ys = _grouped_mlp(xs, W1, W2, gate_padded, block_expert, block_valid,
                      block_clamped, B, 1024, interpret=interpret)
    if interpret:
        y = ys[pos_tk[:, 0]]
        for k in range(1, K):
            y = y + ys[pos_tk[:, k]]
    else:
        pint = pos_tk.reshape(-1)
        y = _sc_combine(ys, pint, T)
    return y.reshape(*x_shape[:-1], H)


def kernel(x, routing_weights, selected_experts, W1, W2):
    return _fused_experts(x, routing_weights, selected_experts, W1, W2)


# speedup vs baseline: 1.1506x; 1.1506x over previous
import functools

import jax
import jax.numpy as jnp
from jax import lax
from jax.experimental import pallas as pl
from jax.experimental.pallas import tpu as pltpu
from jax.experimental.pallas import tpu_sc as plsc

_SC_CORES = 2
_SC_SUBCORES = 16


def _routing_metadata(selected_experts, routing_weights, E, B, NB):
    T, K = selected_experts.shape
    S = T * K
    flat_e = selected_experts.reshape(-1)
    gates = routing_weights.reshape(-1).astype(jnp.float32)
    onehot = (flat_e[:, None] == jnp.arange(E, dtype=jnp.int32)[None, :])
    counts = jnp.sum(onehot.astype(jnp.int32), axis=0)
    pc = ((counts + B - 1) // B) * B
    ends = jnp.cumsum(pc)
    starts = ends - pc
    rank_all = jnp.cumsum(onehot.astype(jnp.int32), axis=0) - 1
    rank = jnp.take_along_axis(rank_all, flat_e[:, None], axis=1)[:, 0]
    pos = starts[flat_e] + rank
    token_ids = (jnp.arange(S, dtype=jnp.int32) // K)
    NP = NB * B
    tok_padded = jnp.zeros((NP,), jnp.int32).at[pos].set(token_ids)
    gate_padded = jnp.zeros((NP,), jnp.float32).at[pos].set(gates)
    brow = jnp.arange(NB, dtype=jnp.int32) * B
    block_expert = jnp.sum((brow[:, None] >= ends[None, :]).astype(jnp.int32),
                           axis=1)
    block_expert = jnp.minimum(block_expert, E - 1)
    nact = ends[E - 1] // B
    block_valid = (jnp.arange(NB, dtype=jnp.int32) < nact).astype(jnp.int32)
    block_clamped = jnp.minimum(jnp.arange(NB, dtype=jnp.int32), nact - 1)
    pos_tk = pos.reshape(T, K)
    return tok_padded, gate_padded, block_expert, block_valid, block_clamped, pos_tk


def _sc_scatter(xf, pint3, NP):
    T, H = xf.shape
    NW = _SC_CORES * _SC_SUBCORES
    TW = T // NW
    K = pint3.shape[1]
    mesh = plsc.VectorSubcoreMesh(core_axis_name="c", subcore_axis_name="s")

    @functools.partial(
        pl.kernel, mesh=mesh,
        out_type=jax.ShapeDtypeStruct((NP, H), jnp.float32),
        scratch_types=[
            pltpu.VMEM((K, TW), jnp.int32),
            pltpu.VMEM((TW, H), jnp.float32),
        ] + [pltpu.SemaphoreType.DMA] * K,
    )
    def k(x_hbm, pint_hbm, out_hbm, idx_v, src_v, *sems):
        wid = lax.axis_index("s") * _SC_CORES + lax.axis_index("c")
        base = wid * TW
        pltpu.sync_copy(pint_hbm.at[wid], idx_v)
        pltpu.sync_copy(x_hbm.at[pl.ds(base, TW)], src_v)
        descs = [
            pltpu.async_copy(src_v, out_hbm.at[idx_v.at[kk]], sems[kk])
            for kk in range(K)
        ]
        for d in descs:
            d.wait()

    return k(xf, pint3)


def _sc_combine(ys, pint, T):
    H = ys.shape[1]
    NW = _SC_CORES * _SC_SUBCORES
    tok_w = T // NW
    CT = 16
    n_ch = tok_w // CT
    mesh = plsc.VectorSubcoreMesh(core_axis_name="c", subcore_axis_name="s")

    @functools.partial(
        pl.kernel, mesh=mesh,
        out_type=jax.ShapeDtypeStruct((T, H), jnp.float32),
        scratch_types=[
            pltpu.VMEM((2 * tok_w,), jnp.int32),
            pltpu.VMEM((2 * CT, H), jnp.float32),
            pltpu.VMEM((2 * CT, H), jnp.float32),
            pltpu.VMEM((CT, H), jnp.float32),
            pltpu.SemaphoreType.DMA,
            pltpu.SemaphoreType.DMA,
        ],
    )
    def k(ys_hbm, pint_hbm, y_hbm, idx_v, bufa, bufb, outb, sem0, sem1):
        wid = lax.axis_index("s") * _SC_CORES + lax.axis_index("c")
        tok_base = wid * tok_w
        pltpu.sync_copy(pint_hbm.at[pl.ds(2 * tok_base, 2 * tok_w)], idx_v)
        bufs = (bufa, bufb)
        sems = (sem0, sem1)
        descs = [None] * n_ch
        descs[0] = pltpu.async_copy(
            ys_hbm.at[idx_v.at[pl.ds(0, 2 * CT)]], bufs[0], sems[0])
        nvec = H // 16
        for c in range(n_ch):
            if c + 1 < n_ch:
                descs[c + 1] = pltpu.async_copy(
                    ys_hbm.at[idx_v.at[pl.ds((c + 1) * 2 * CT, 2 * CT)]],
                    bufs[(c + 1) % 2], sems[(c + 1) % 2])
            descs[c].wait()
            buf = bufs[c % 2]

            def body(t, _, buf=buf):
                for j in range(nvec):
                    off = j * 16
                    outb[t, pl.ds(off, 16)] = (
                        buf[2 * t, pl.ds(off, 16)]
                        + buf[2 * t + 1, pl.ds(off, 16)])
                return 0

            lax.fori_loop(0, CT, body, 0)
            pltpu.sync_copy(outb, y_hbm.at[pl.ds(tok_base + c * CT, CT)])

    return k(ys, pint)


def _gmm_body(NF, be_ref, bv_ref, bc_ref, xs_ref, w1h_ref, w1g_ref, w2_ref,
              g_ref, out_ref):
    b = pl.program_id(0)
    f = pl.program_id(1)

    @pl.when(bv_ref[b] > 0)
    def _():
        @pl.when(f == 0)
        def _():
            out_ref[...] = jnp.zeros_like(out_ref)

        x = xs_ref[...]
        h = lax.dot_general(x, w1h_ref[0], (((1,), (1,)), ((), ())),
                            preferred_element_type=jnp.float32)
        g = lax.dot_general(x, w1g_ref[0], (((1,), (1,)), ((), ())),
                            preferred_element_type=jnp.float32)
        hg = h * (g * jax.nn.sigmoid(g))
        yp = lax.dot_general(hg, w2_ref[0], (((1,), (1,)), ((), ())),
                             preferred_element_type=jnp.float32)
        out_ref[...] += yp

        @pl.when(f == NF - 1)
        def _():
            out_ref[...] *= g_ref[0, 0][:, None]


def _grouped_mlp(xs, w1, w2, gate_padded, block_expert, block_valid,
                 block_clamped, B, TF, interpret=False):
    E, F2, H = w1.shape
    F = F2 // 2
    NF = F // TF
    NP = xs.shape[0]
    NB = NP // B
    gates3 = gate_padded.reshape(NB, 1, B)

    def f_eff(f, bv, b):
        return bv[b] * f + (1 - bv[b]) * (NF - 1)

    grid_spec = pltpu.PrefetchScalarGridSpec(
        num_scalar_prefetch=3,
        grid=(NB, NF),
        in_specs=[
            pl.BlockSpec((B, H), lambda b, f, be, bv, bc: (bc[b], 0)),
            pl.BlockSpec((1, TF, H),
                         lambda b, f, be, bv, bc:
                         (be[bc[b]], f_eff(f, bv, b), 0)),
            pl.BlockSpec((1, TF, H),
                         lambda b, f, be, bv, bc:
                         (be[bc[b]], NF + f_eff(f, bv, b), 0)),
            pl.BlockSpec((1, H, TF),
                         lambda b, f, be, bv, bc:
                         (be[bc[b]], 0, f_eff(f, bv, b))),
            pl.BlockSpec((1, 1, B), lambda b, f, be, bv, bc: (bc[b], 0, 0)),
        ],
        out_specs=pl.BlockSpec((B, H), lambda b, f, be, bv, bc: (bc[b], 0)),
    )
    return pl.pallas_call(
        functools.partial(_gmm_body, NF),
        grid_spec=grid_spec,
        out_shape=jax.ShapeDtypeStruct((NP, H), jnp.float32),
        compiler_params=pltpu.CompilerParams(
            dimension_semantics=("arbitrary", "arbitrary")),
        interpret=interpret,
    )(block_expert, block_valid, block_clamped, xs, w1, w1, w2, gates3)


def _fused_experts(x, routing_weights, selected_experts, W1, W2,
                   interpret=False):
    x_shape = x.shape
    H = x_shape[-1]
    xf = x.reshape(-1, H)
    T, K = selected_experts.shape
    E = W1.shape[0]
    B = 768
    S = T * K
    NB = (S + E * (B - 1) + B - 1) // B
    tok_padded, gate_padded, block_expert, block_valid, block_clamped, pos_tk = \
        _routing_metadata(selected_experts, routing_weights, E, B, NB)
    NW = _SC_CORES * _SC_SUBCORES
    if interpret:
        xs = jnp.take(xf, tok_padded, axis=0)
    else:
        pint3 = pos_tk.reshape(NW, T // NW, K).transpose(0, 2, 1)
        xs = _sc_scatter(xf, pint3, NB * B)
    ys = _grouped_mlp(xs, W1, W2, gate_padded, block_expert, block_valid,
                      block_clamped, B, 1024, interpret=interpret)
    if interpret:
        y = ys[pos_tk[:, 0]]
        for k in range(1, K):
            y = y + ys[pos_tk[:, k]]
    else:
        pint = pos_tk.reshape(-1)
        y = _sc_combine(ys, pint, T)
    return y.reshape(*x_shape[:-1], H)


def kernel(x, routing_weights, selected_experts, W1, W2):
    return _fused_experts(x, routing_weights, selected_experts, W1, W2)


# speedup vs baseline: 1.1526x; 1.0017x over previous
import functools

import jax
import jax.numpy as jnp
from jax import lax
from jax.experimental import pallas as pl
from jax.experimental.pallas import tpu as pltpu
from jax.experimental.pallas import tpu_sc as plsc

_SC_CORES = 2
_SC_SUBCORES = 16


def _routing_metadata(selected_experts, routing_weights, E, B, NB):
    T, K = selected_experts.shape
    S = T * K
    flat_e = selected_experts.reshape(-1)
    gates = routing_weights.reshape(-1).astype(jnp.float32)
    onehot = (flat_e[:, None] == jnp.arange(E, dtype=jnp.int32)[None, :])
    counts = jnp.sum(onehot.astype(jnp.int32), axis=0)
    pc = ((counts + B - 1) // B) * B
    ends = jnp.cumsum(pc)
    starts = ends - pc
    rank_all = jnp.cumsum(onehot.astype(jnp.int32), axis=0) - 1
    rank = jnp.take_along_axis(rank_all, flat_e[:, None], axis=1)[:, 0]
    pos = starts[flat_e] + rank
    NP = NB * B
    gate_padded = jnp.zeros((NP,), jnp.float32).at[pos].set(gates)
    brow = jnp.arange(NB, dtype=jnp.int32) * B
    block_expert = jnp.sum((brow[:, None] >= ends[None, :]).astype(jnp.int32),
                           axis=1)
    block_expert = jnp.minimum(block_expert, E - 1)
    nact = ends[E - 1] // B
    block_valid = (jnp.arange(NB, dtype=jnp.int32) < nact).astype(jnp.int32)
    block_clamped = jnp.minimum(jnp.arange(NB, dtype=jnp.int32), nact - 1)
    pos_tk = pos.reshape(T, K)
    return gate_padded, block_expert, block_valid, block_clamped, pos_tk


def _sc_scatter(xf, pint3, NP):
    T, H = xf.shape
    NW = _SC_CORES * _SC_SUBCORES
    TW = T // NW
    K = pint3.shape[1]
    mesh = plsc.VectorSubcoreMesh(core_axis_name="c", subcore_axis_name="s")

    @functools.partial(
        pl.kernel, mesh=mesh,
        out_type=jax.ShapeDtypeStruct((NP, H), jnp.float32),
        scratch_types=[
            pltpu.VMEM((K, TW), jnp.int32),
            pltpu.VMEM((TW, H), jnp.float32),
        ] + [pltpu.SemaphoreType.DMA] * K,
    )
    def k(x_hbm, pint_hbm, out_hbm, idx_v, src_v, *sems):
        wid = lax.axis_index("s") * _SC_CORES + lax.axis_index("c")
        base = wid * TW
        pltpu.sync_copy(pint_hbm.at[wid], idx_v)
        pltpu.sync_copy(x_hbm.at[pl.ds(base, TW)], src_v)
        descs = [
            pltpu.async_copy(src_v, out_hbm.at[idx_v.at[kk]], sems[kk])
            for kk in range(K)
        ]
        for d in descs:
            d.wait()

    return k(xf, pint3)


def _sc_combine(ys, pint, T):
    H = ys.shape[1]
    NW = _SC_CORES * _SC_SUBCORES
    tok_w = T // NW
    CT = 16
    n_ch = tok_w // CT
    mesh = plsc.VectorSubcoreMesh(core_axis_name="c", subcore_axis_name="s")

    @functools.partial(
        pl.kernel, mesh=mesh,
        out_type=jax.ShapeDtypeStruct((T, H), jnp.float32),
        scratch_types=[
            pltpu.VMEM((2 * tok_w,), jnp.int32),
            pltpu.VMEM((2 * CT, H), jnp.float32),
            pltpu.VMEM((2 * CT, H), jnp.float32),
            pltpu.VMEM((CT, H), jnp.float32),
            pltpu.SemaphoreType.DMA,
            pltpu.SemaphoreType.DMA,
        ],
    )
    def k(ys_hbm, pint_hbm, y_hbm, idx_v, bufa, bufb, outb, sem0, sem1):
        wid = lax.axis_index("s") * _SC_CORES + lax.axis_index("c")
        tok_base = wid * tok_w
        pltpu.sync_copy(pint_hbm.at[pl.ds(2 * tok_base, 2 * tok_w)], idx_v)
        bufs = (bufa, bufb)
        sems = (sem0, sem1)
        descs = [None] * n_ch
        descs[0] = pltpu.async_copy(
            ys_hbm.at[idx_v.at[pl.ds(0, 2 * CT)]], bufs[0], sems[0])
        nvec = H // 16
        for c in range(n_ch):
            if c + 1 < n_ch:
                descs[c + 1] = pltpu.async_copy(
                    ys_hbm.at[idx_v.at[pl.ds((c + 1) * 2 * CT, 2 * CT)]],
                    bufs[(c + 1) % 2], sems[(c + 1) % 2])
            descs[c].wait()
            buf = bufs[c % 2]

            def body(t, _, buf=buf):
                for j in range(nvec):
                    off = j * 16
                    outb[t, pl.ds(off, 16)] = (
                        buf[2 * t, pl.ds(off, 16)]
                        + buf[2 * t + 1, pl.ds(off, 16)])
                return 0

            lax.fori_loop(0, CT, body, 0)
            pltpu.sync_copy(outb, y_hbm.at[pl.ds(tok_base + c * CT, CT)])

    return k(ys, pint)


def _gmm_body(NF, be_ref, bv_ref, bc_ref, xs_ref, w1h_ref, w1g_ref, w2_ref,
              g_ref, out_ref):
    b = pl.program_id(0)
    f = pl.program_id(1)

    @pl.when(bv_ref[b] > 0)
    def _():
        @pl.when(f == 0)
        def _():
            out_ref[...] = jnp.zeros_like(out_ref)

        x = xs_ref[...]
        h = lax.dot_general(x, w1h_ref[0], (((1,), (1,)), ((), ())),
                            preferred_element_type=jnp.float32)
        g = lax.dot_general(x, w1g_ref[0], (((1,), (1,)), ((), ())),
                            preferred_element_type=jnp.float32)
        hg = h * (g * jax.nn.sigmoid(g))
        yp = lax.dot_general(hg, w2_ref[0], (((1,), (1,)), ((), ())),
                             preferred_element_type=jnp.float32)
        out_ref[...] += yp

        @pl.when(f == NF - 1)
        def _():
            out_ref[...] *= g_ref[0, 0][:, None]


def _grouped_mlp(xs, w1, w2, gate_padded, block_expert, block_valid,
                 block_clamped, B, TF):
    E, F2, H = w1.shape
    F = F2 // 2
    NF = F // TF
    NP = xs.shape[0]
    NB = NP // B
    gates3 = gate_padded.reshape(NB, 1, B)

    def f_eff(f, bv, b):
        return bv[b] * f + (1 - bv[b]) * (NF - 1)

    grid_spec = pltpu.PrefetchScalarGridSpec(
        num_scalar_prefetch=3,
        grid=(NB, NF),
        in_specs=[
            pl.BlockSpec((B, H), lambda b, f, be, bv, bc: (bc[b], 0)),
            pl.BlockSpec((1, TF, H),
                         lambda b, f, be, bv, bc:
                         (be[bc[b]], f_eff(f, bv, b), 0)),
            pl.BlockSpec((1, TF, H),
                         lambda b, f, be, bv, bc:
                         (be[bc[b]], NF + f_eff(f, bv, b), 0)),
            pl.BlockSpec((1, H, TF),
                         lambda b, f, be, bv, bc:
                         (be[bc[b]], 0, f_eff(f, bv, b))),
            pl.BlockSpec((1, 1, B), lambda b, f, be, bv, bc: (bc[b], 0, 0)),
        ],
        out_specs=pl.BlockSpec((B, H), lambda b, f, be, bv, bc: (bc[b], 0)),
    )
    return pl.pallas_call(
        functools.partial(_gmm_body, NF),
        grid_spec=grid_spec,
        out_shape=jax.ShapeDtypeStruct((NP, H), jnp.float32),
        compiler_params=pltpu.CompilerParams(
            dimension_semantics=("arbitrary", "arbitrary")),
    )(block_expert, block_valid, block_clamped, xs, w1, w1, w2, gates3)


def kernel(x, routing_weights, selected_experts, W1, W2):
    x_shape = x.shape
    H = x_shape[-1]
    xf = x.reshape(-1, H)
    T, K = selected_experts.shape
    E = W1.shape[0]
    B = 768
    S = T * K
    NB = (S + E * (B - 1) + B - 1) // B
    gate_padded, block_expert, block_valid, block_clamped, pos_tk = \
        _routing_metadata(selected_experts, routing_weights, E, B, NB)
    NW = _SC_CORES * _SC_SUBCORES
    pint3 = pos_tk.reshape(NW, T // NW, K).transpose(0, 2, 1)
    xs = _sc_scatter(xf, pint3, NB * B)
    ys = _grouped_mlp(xs, W1, W2, gate_padded, block_expert, block_valid,
                      block_clamped, B, 1024)
    pint = pos_tk.reshape(-1)
    y = _sc_combine(ys, pint, T)
    return y.reshape(*x_shape[:-1], H)
